# Initial kernel scaffold; baseline (speedup 1.0000x reference)
import functools

import jax
import jax.numpy as jnp
from jax import lax
from jax.experimental import pallas as pl
from jax.experimental.pallas import tpu as pltpu
from jax.experimental.pallas import tpu_sc as plsc

N = 10000
E = 320000
D = 128

NC = 2
NS = 16
NW = NC * NS

NP = 10240
STRIPE = NP // NS

EPW = E // NW
CHUNK = 80
NCHUNK = EPW // CHUNK
EROWS = E // CHUNK

_mesh = plsc.VectorSubcoreMesh(
    core_axis_name="c", subcore_axis_name="s", num_cores=NC, num_subcores=NS
)


def _worker_id():
    return lax.axis_index("c") * NS + lax.axis_index("s")


@functools.partial(
    pl.kernel,
    out_type=jax.ShapeDtypeStruct((NW, NP), jnp.float32),
    mesh=_mesh,
    scratch_types=[
        pltpu.VMEM((EPW,), jnp.int32),
        pltpu.VMEM((NP,), jnp.float32),
    ],
)
def _sc_degree(dst_hbm, out_hbm, didx_v, hist_v):
    wid = _worker_id()
    zeros = jnp.zeros((16,), jnp.float32)
    ones = jnp.ones((16,), jnp.float32)

    @pl.loop(0, NP // 16)
    def _zero(t):
        hist_v[pl.ds(t * 16, 16)] = zeros

    pltpu.sync_copy(dst_hbm.at[pl.ds(wid * EPW, EPW)], didx_v)

    @pl.loop(0, EPW // 16)
    def _accum(t):
        idx = didx_v[pl.ds(t * 16, 16)]
        plsc.addupdate_scatter(hist_v, [idx], ones)

    pltpu.sync_copy(hist_v, out_hbm.at[wid])


@functools.partial(
    pl.kernel,
    out_type=jax.ShapeDtypeStruct((NC, NP, D), jnp.float32),
    mesh=_mesh,
    scratch_types=[
        pltpu.VMEM((NCHUNK, CHUNK), jnp.int32),
        pltpu.VMEM((NCHUNK, CHUNK), jnp.int32),
        pltpu.VMEM((CHUNK, D), jnp.float32),
        pltpu.VMEM_SHARED((NP, D), jnp.float32),
        pltpu.SemaphoreType.DMA,
    ],
)
def _sc_scatter(hs_hbm, src2d_hbm, dst2d_hbm, zeros_hbm, out_hbm,
                sidx_v, didx_v, rows_v, acc_sh, sem):
    cid = lax.axis_index("c")
    sid = lax.axis_index("s")
    wid = cid * NS + sid

    pltpu.sync_copy(zeros_hbm.at[pl.ds(sid * STRIPE, STRIPE)],
                    acc_sh.at[pl.ds(sid * STRIPE, STRIPE)])

    pltpu.sync_copy(src2d_hbm.at[pl.ds(wid * NCHUNK, NCHUNK)], sidx_v)
    pltpu.sync_copy(dst2d_hbm.at[pl.ds(wid * NCHUNK, NCHUNK)], didx_v)

    plsc.subcore_barrier()

    @pl.loop(0, NCHUNK)
    def _edges(j):
        pltpu.async_copy(hs_hbm.at[sidx_v.at[j]], rows_v, sem).wait()
        pltpu.sync_copy(rows_v, acc_sh.at[didx_v.at[j]], add=True)

    plsc.subcore_barrier()

    pltpu.sync_copy(acc_sh.at[pl.ds(sid * STRIPE, STRIPE)],
                    out_hbm.at[cid, pl.ds(sid * STRIPE, STRIPE)])


RB = 500
NG = N // RB


def _tc_prep_body(deg_ref, x_ref, w_ref, b_ref, dis_ref, hs_ref):
    cnt = jnp.sum(deg_ref[...], axis=0)
    dis = lax.rsqrt(cnt + 1.0).reshape(RB, 1)
    dis_ref[...] = dis
    h = jnp.dot(x_ref[...], w_ref[...], preferred_element_type=jnp.float32)
    hs_ref[...] = (h + b_ref[...]) * dis


def _tc_prep(deg_p, x, w_enc, b_enc):
    return pl.pallas_call(
        _tc_prep_body,
        grid=(NG,),
        in_specs=[
            pl.BlockSpec((NW, RB), lambda i: (0, i)),
            pl.BlockSpec((RB, D), lambda i: (i, 0)),
            pl.BlockSpec((D, D), lambda i: (0, 0)),
            pl.BlockSpec((1, D), lambda i: (0, 0)),
        ],
        out_specs=[
            pl.BlockSpec((RB, 1), lambda i: (i, 0)),
            pl.BlockSpec((RB, D), lambda i: (i, 0)),
        ],
        out_shape=[
            jax.ShapeDtypeStruct((N, 1), jnp.float32),
            jax.ShapeDtypeStruct((N, D), jnp.float32),
        ],
    )(deg_p, x, w_enc, b_enc)


def _tc_mid_body(p0_ref, p1_ref, hs_ref, dis_ref, w_ref, b_ref, out_ref):
    agg = (p0_ref[...] + p1_ref[...] + hs_ref[...]) * dis_ref[...]
    h = jnp.dot(agg, w_ref[...], preferred_element_type=jnp.float32)
    out_ref[...] = jax.nn.relu(h + b_ref[...]) * dis_ref[...]


def _tc_mid(p0, p1, hs, dis, w, b):
    return pl.pallas_call(
        _tc_mid_body,
        grid=(NG,),
        in_specs=[
            pl.BlockSpec((RB, D), lambda i: (i, 0)),
            pl.BlockSpec((RB, D), lambda i: (i, 0)),
            pl.BlockSpec((RB, D), lambda i: (i, 0)),
            pl.BlockSpec((RB, 1), lambda i: (i, 0)),
            pl.BlockSpec((D, D), lambda i: (0, 0)),
            pl.BlockSpec((1, D), lambda i: (0, 0)),
        ],
        out_specs=pl.BlockSpec((RB, D), lambda i: (i, 0)),
        out_shape=jax.ShapeDtypeStruct((N, D), jnp.float32),
    )(p0, p1, hs, dis, w, b)


def _tc_final_body(p0_ref, p1_ref, hs_ref, dis_ref, w_ref, b_ref,
                   wh_ref, bh_ref, out_ref):
    agg = (p0_ref[...] + p1_ref[...] + hs_ref[...]) * dis_ref[...]
    h = jnp.dot(agg, w_ref[...], preferred_element_type=jnp.float32) + b_ref[...]
    out_ref[...] = (
        jnp.dot(h, wh_ref[...], preferred_element_type=jnp.float32) + bh_ref[...]
    )


def _tc_final(p0, p1, hs, dis, w, b, wh, bh):
    return pl.pallas_call(
        _tc_final_body,
        grid=(NG,),
        in_specs=[
            pl.BlockSpec((RB, D), lambda i: (i, 0)),
            pl.BlockSpec((RB, D), lambda i: (i, 0)),
            pl.BlockSpec((RB, D), lambda i: (i, 0)),
            pl.BlockSpec((RB, 1), lambda i: (i, 0)),
            pl.BlockSpec((D, D), lambda i: (0, 0)),
            pl.BlockSpec((1, D), lambda i: (0, 0)),
            pl.BlockSpec((D, D), lambda i: (0, 0)),
            pl.BlockSpec((1, D), lambda i: (0, 0)),
        ],
        out_specs=pl.BlockSpec((RB, D), lambda i: (i, 0)),
        out_shape=jax.ShapeDtypeStruct((N, D), jnp.float32),
    )(p0, p1, hs, dis, w, b, wh, bh)


@jax.jit
def kernel(x, edge_index, W_enc, b_enc, W1, b1, W2, b2, W3, b3, W_head, b_head):
    src2d = edge_index[0].reshape(EROWS, CHUNK)
    dst2d = edge_index[1].reshape(EROWS, CHUNK)
    dst1d = edge_index[1]
    zeros = jnp.zeros((NP, D), jnp.float32)

    deg_p = _sc_degree(dst1d)

    dis, hs = _tc_prep(deg_p, x, W_enc, b_enc.reshape(1, D))

    def layer_parts(hs_k):
        p = _sc_scatter(hs_k, src2d, dst2d, zeros)
        return p[0, :N], p[1, :N]

    p0, p1 = layer_parts(hs)
    hs = _tc_mid(p0, p1, hs, dis, W1, b1.reshape(1, D))
    p0, p1 = layer_parts(hs)
    hs = _tc_mid(p0, p1, hs, dis, W2, b2.reshape(1, D))
    p0, p1 = layer_parts(hs)
    out = _tc_final(p0, p1, hs, dis, W3, b3.reshape(1, D),
                    W_head, b_head.reshape(1, D))
    return out


# speedup vs baseline: 5.6850x; 5.6850x over previous
import functools

import jax
import jax.numpy as jnp
from jax import lax
from jax.experimental import pallas as pl
from jax.experimental.pallas import tpu as pltpu
from jax.experimental.pallas import tpu_sc as plsc

N = 10000
E = 320000
D = 128

NC = 2
NS = 16
NW = NC * NS

NP = 10240
STRIPE = NP // NS

CHUNK = 80
NCHUNK = 128
EROWS = NW * NCHUNK
EPAD = EROWS * CHUNK

_mesh = plsc.VectorSubcoreMesh(
    core_axis_name="c", subcore_axis_name="s", num_cores=NC, num_subcores=NS
)


def _worker_id():
    return lax.axis_index("c") * NS + lax.axis_index("s")


@functools.partial(
    pl.kernel,
    out_type=jax.ShapeDtypeStruct((NC, NP), jnp.float32),
    mesh=_mesh,
    scratch_types=[
        pltpu.VMEM((NCHUNK, CHUNK), jnp.int32),
        pltpu.VMEM((CHUNK,), jnp.float32),
        pltpu.VMEM_SHARED((NP,), jnp.float32),
    ],
)
def _sc_degree(dst2d_hbm, zeros1_hbm, out_hbm, didx_v, ones_v, deg_sh):
    cid = lax.axis_index("c")
    sid = lax.axis_index("s")
    wid = cid * NS + sid
    ones = jnp.ones((16,), jnp.float32)

    @pl.loop(0, CHUNK // 16)
    def _fill(t):
        ones_v[pl.ds(t * 16, 16)] = ones

    pltpu.sync_copy(zeros1_hbm.at[pl.ds(sid * STRIPE, STRIPE)],
                    deg_sh.at[pl.ds(sid * STRIPE, STRIPE)])
    pltpu.sync_copy(dst2d_hbm.at[pl.ds(wid * NCHUNK, NCHUNK)], didx_v)

    plsc.subcore_barrier()

    @pl.loop(0, NCHUNK)
    def _accum(j):
        pltpu.sync_copy(ones_v, deg_sh.at[didx_v.at[j]], add=True)

    plsc.subcore_barrier()

    pltpu.sync_copy(deg_sh.at[pl.ds(sid * STRIPE, STRIPE)],
                    out_hbm.at[cid, pl.ds(sid * STRIPE, STRIPE)])


@functools.partial(
    pl.kernel,
    out_type=jax.ShapeDtypeStruct((NC, NP, D), jnp.float32),
    mesh=_mesh,
    scratch_types=[
        pltpu.VMEM((NCHUNK, CHUNK), jnp.int32),
        pltpu.VMEM((NCHUNK, CHUNK), jnp.int32),
        pltpu.VMEM((CHUNK, D), jnp.float32),
        pltpu.VMEM_SHARED((NP, D), jnp.float32),
        pltpu.SemaphoreType.DMA,
    ],
)
def _sc_scatter(hs_hbm, src2d_hbm, dst2d_hbm, zeros_hbm, out_hbm,
                sidx_v, didx_v, rows_v, acc_sh, sem):
    cid = lax.axis_index("c")
    sid = lax.axis_index("s")
    wid = cid * NS + sid

    pltpu.sync_copy(zeros_hbm.at[pl.ds(sid * STRIPE, STRIPE)],
                    acc_sh.at[pl.ds(sid * STRIPE, STRIPE)])

    pltpu.sync_copy(src2d_hbm.at[pl.ds(wid * NCHUNK, NCHUNK)], sidx_v)
    pltpu.sync_copy(dst2d_hbm.at[pl.ds(wid * NCHUNK, NCHUNK)], didx_v)

    plsc.subcore_barrier()

    @pl.loop(0, NCHUNK)
    def _edges(j):
        pltpu.async_copy(hs_hbm.at[sidx_v.at[j]], rows_v, sem).wait()
        pltpu.sync_copy(rows_v, acc_sh.at[didx_v.at[j]], add=True)

    plsc.subcore_barrier()

    pltpu.sync_copy(acc_sh.at[pl.ds(sid * STRIPE, STRIPE)],
                    out_hbm.at[cid, pl.ds(sid * STRIPE, STRIPE)])


RB = 512
NG = NP // RB


def _tc_prep_body(deg_ref, x_ref, w_ref, b_ref, dis_ref, hs_ref):
    cnt = jnp.sum(deg_ref[...], axis=0)
    dis = lax.rsqrt(cnt + 1.0).reshape(RB, 1)
    dis_ref[...] = dis
    h = jnp.dot(x_ref[...], w_ref[...], preferred_element_type=jnp.float32)
    hs_ref[...] = (h + b_ref[...]) * dis


def _tc_prep(deg_p, x, w_enc, b_enc):
    return pl.pallas_call(
        _tc_prep_body,
        grid=(NG,),
        in_specs=[
            pl.BlockSpec((NC, RB), lambda i: (0, i)),
            pl.BlockSpec((RB, D), lambda i: (i, 0)),
            pl.BlockSpec((D, D), lambda i: (0, 0)),
            pl.BlockSpec((1, D), lambda i: (0, 0)),
        ],
        out_specs=[
            pl.BlockSpec((RB, 1), lambda i: (i, 0)),
            pl.BlockSpec((RB, D), lambda i: (i, 0)),
        ],
        out_shape=[
            jax.ShapeDtypeStruct((NP, 1), jnp.float32),
            jax.ShapeDtypeStruct((NP, D), jnp.float32),
        ],
    )(deg_p, x, w_enc, b_enc)


def _tc_mid_body(p0_ref, p1_ref, hs_ref, dis_ref, w_ref, b_ref, out_ref):
    agg = (p0_ref[...] + p1_ref[...] + hs_ref[...]) * dis_ref[...]
    h = jnp.dot(agg, w_ref[...], preferred_element_type=jnp.float32)
    out_ref[...] = jax.nn.relu(h + b_ref[...]) * dis_ref[...]


def _tc_mid(p0, p1, hs, dis, w, b):
    return pl.pallas_call(
        _tc_mid_body,
        grid=(NG,),
        in_specs=[
            pl.BlockSpec((RB, D), lambda i: (i, 0)),
            pl.BlockSpec((RB, D), lambda i: (i, 0)),
            pl.BlockSpec((RB, D), lambda i: (i, 0)),
            pl.BlockSpec((RB, 1), lambda i: (i, 0)),
            pl.BlockSpec((D, D), lambda i: (0, 0)),
            pl.BlockSpec((1, D), lambda i: (0, 0)),
        ],
        out_specs=pl.BlockSpec((RB, D), lambda i: (i, 0)),
        out_shape=jax.ShapeDtypeStruct((NP, D), jnp.float32),
    )(p0, p1, hs, dis, w, b)


def _tc_final_body(p0_ref, p1_ref, hs_ref, dis_ref, w_ref, b_ref,
                   wh_ref, bh_ref, out_ref):
    agg = (p0_ref[...] + p1_ref[...] + hs_ref[...]) * dis_ref[...]
    h = jnp.dot(agg, w_ref[...], preferred_element_type=jnp.float32) + b_ref[...]
    out_ref[...] = (
        jnp.dot(h, wh_ref[...], preferred_element_type=jnp.float32) + bh_ref[...]
    )


def _tc_final(p0, p1, hs, dis, w, b, wh, bh):
    return pl.pallas_call(
        _tc_final_body,
        grid=(NG,),
        in_specs=[
            pl.BlockSpec((RB, D), lambda i: (i, 0)),
            pl.BlockSpec((RB, D), lambda i: (i, 0)),
            pl.BlockSpec((RB, D), lambda i: (i, 0)),
            pl.BlockSpec((RB, 1), lambda i: (i, 0)),
            pl.BlockSpec((D, D), lambda i: (0, 0)),
            pl.BlockSpec((1, D), lambda i: (0, 0)),
            pl.BlockSpec((D, D), lambda i: (0, 0)),
            pl.BlockSpec((1, D), lambda i: (0, 0)),
        ],
        out_specs=pl.BlockSpec((RB, D), lambda i: (i, 0)),
        out_shape=jax.ShapeDtypeStruct((NP, D), jnp.float32),
    )(p0, p1, hs, dis, w, b, wh, bh)


@jax.jit
def kernel(x, edge_index, W_enc, b_enc, W1, b1, W2, b2, W3, b3, W_head, b_head):
    e_pad = jnp.pad(edge_index, ((0, 0), (0, EPAD - E)), constant_values=N)
    src2d = e_pad[0].reshape(EROWS, CHUNK)
    dst2d = e_pad[1].reshape(EROWS, CHUNK)
    zeros = jnp.zeros((NP, D), jnp.float32)
    zeros1 = jnp.zeros((NP,), jnp.float32)
    x_pad = jnp.pad(x, ((0, NP - N), (0, 0)))

    deg_p = _sc_degree(dst2d, zeros1)

    dis, hs = _tc_prep(deg_p, x_pad, W_enc, b_enc.reshape(1, D))

    def layer_parts(hs_k):
        p = _sc_scatter(hs_k, src2d, dst2d, zeros)
        return p[0], p[1]

    p0, p1 = layer_parts(hs)
    hs = _tc_mid(p0, p1, hs, dis, W1, b1.reshape(1, D))
    p0, p1 = layer_parts(hs)
    hs = _tc_mid(p0, p1, hs, dis, W2, b2.reshape(1, D))
    p0, p1 = layer_parts(hs)
    out = _tc_final(p0, p1, hs, dis, W3, b3.reshape(1, D),
                    W_head, b_head.reshape(1, D))
    return out[:N]


# speedup vs baseline: 6.6704x; 1.1733x over previous
import functools

import jax
import jax.numpy as jnp
from jax import lax
from jax.experimental import pallas as pl
from jax.experimental.pallas import tpu as pltpu
from jax.experimental.pallas import tpu_sc as plsc

N = 10000
E = 320000
D = 128
DH = D // 2

NC = 2
NS = 16
NW = NC * NS

NP = 10240
STRIPE = NP // NS

CHUNK = 128
NCHUNK = 80
HALF = NCHUNK // 2
EROWS = NW * NCHUNK
EPAD = EROWS * CHUNK

_mesh = plsc.VectorSubcoreMesh(
    core_axis_name="c", subcore_axis_name="s", num_cores=NC, num_subcores=NS
)


@functools.partial(
    pl.kernel,
    out_type=jax.ShapeDtypeStruct((NC, NP), jnp.float32),
    mesh=_mesh,
    scratch_types=[
        pltpu.VMEM((NCHUNK, CHUNK), jnp.int32),
        pltpu.VMEM((CHUNK,), jnp.float32),
        pltpu.VMEM_SHARED((NP,), jnp.float32),
    ],
)
def _sc_degree(dst2d_hbm, zeros1_hbm, out_hbm, didx_v, ones_v, deg_sh):
    cid = lax.axis_index("c")
    sid = lax.axis_index("s")
    wid = cid * NS + sid
    ones = jnp.ones((16,), jnp.float32)

    @pl.loop(0, CHUNK // 16)
    def _fill(t):
        ones_v[pl.ds(t * 16, 16)] = ones

    pltpu.sync_copy(zeros1_hbm.at[pl.ds(sid * STRIPE, STRIPE)],
                    deg_sh.at[pl.ds(sid * STRIPE, STRIPE)])
    pltpu.sync_copy(dst2d_hbm.at[pl.ds(wid * NCHUNK, NCHUNK)], didx_v)

    plsc.subcore_barrier()

    @pl.loop(0, NCHUNK)
    def _accum(j):
        pltpu.sync_copy(ones_v, deg_sh.at[didx_v.at[j]], add=True)

    plsc.subcore_barrier()

    pltpu.sync_copy(deg_sh.at[pl.ds(sid * STRIPE, STRIPE)],
                    out_hbm.at[cid, pl.ds(sid * STRIPE, STRIPE)])


@functools.partial(
    pl.kernel,
    out_type=jax.ShapeDtypeStruct((NC, NP, D), jnp.float32),
    mesh=_mesh,
    scratch_types=[
        pltpu.VMEM((HALF, CHUNK), jnp.int32),
        pltpu.VMEM((HALF, CHUNK), jnp.int32),
        pltpu.VMEM((CHUNK, D), jnp.float32),
        pltpu.VMEM((CHUNK, D), jnp.float32),
        pltpu.VMEM_SHARED((NP, D), jnp.float32),
        pltpu.SemaphoreType.DMA,
        pltpu.SemaphoreType.DMA,
    ],
)
def _sc_scatter(hs_hbm, src2d_hbm, dst2d_hbm, zeros_hbm, out_hbm,
                sidx_v, didx_v, rows0_v, rows1_v, acc_sh, sem0, sem1):
    cid = lax.axis_index("c")
    sid = lax.axis_index("s")
    wid = cid * NS + sid

    pltpu.sync_copy(zeros_hbm.at[pl.ds(sid * STRIPE, STRIPE)],
                    acc_sh.at[pl.ds(sid * STRIPE, STRIPE)])

    plsc.subcore_barrier()

    def _half(h):
        base = wid * NCHUNK + h * HALF
        pltpu.sync_copy(src2d_hbm.at[pl.ds(base, HALF)], sidx_v)
        pltpu.sync_copy(dst2d_hbm.at[pl.ds(base, HALF)], didx_v)

        pltpu.async_copy(hs_hbm.at[sidx_v.at[0]], rows0_v, sem0)

        @pl.loop(0, HALF // 2 - 1)
        def _edges(i):
            j = 2 * i
            pltpu.async_copy(hs_hbm.at[sidx_v.at[j + 1]], rows1_v, sem1)
            pltpu.make_async_copy(hs_hbm.at[sidx_v.at[j]], rows0_v,
                                  sem0).wait()
            pltpu.sync_copy(rows0_v, acc_sh.at[didx_v.at[j]], add=True)
            pltpu.async_copy(hs_hbm.at[sidx_v.at[j + 2]], rows0_v, sem0)
            pltpu.make_async_copy(hs_hbm.at[sidx_v.at[j + 1]], rows1_v,
                                  sem1).wait()
            pltpu.sync_copy(rows1_v, acc_sh.at[didx_v.at[j + 1]], add=True)

        jl = HALF - 2
        pltpu.async_copy(hs_hbm.at[sidx_v.at[jl + 1]], rows1_v, sem1)
        pltpu.make_async_copy(hs_hbm.at[sidx_v.at[jl]], rows0_v, sem0).wait()
        pltpu.sync_copy(rows0_v, acc_sh.at[didx_v.at[jl]], add=True)
        pltpu.make_async_copy(hs_hbm.at[sidx_v.at[jl + 1]], rows1_v,
                              sem1).wait()
        pltpu.sync_copy(rows1_v, acc_sh.at[didx_v.at[jl + 1]], add=True)

    _half(0)
    _half(1)

    plsc.subcore_barrier()

    pltpu.sync_copy(acc_sh.at[pl.ds(sid * STRIPE, STRIPE)],
                    out_hbm.at[cid, pl.ds(sid * STRIPE, STRIPE)])


RB = 512
NG = NP // RB


def _tc_prep_body(deg_ref, x_ref, w_ref, b_ref, dis_ref, hs_ref):
    cnt = jnp.sum(deg_ref[...], axis=0)
    dis = lax.rsqrt(cnt + 1.0).reshape(RB, 1)
    dis_ref[...] = dis
    h = jnp.dot(x_ref[...], w_ref[...], preferred_element_type=jnp.float32)
    hs_ref[...] = (h + b_ref[...]) * dis


def _tc_prep(deg_p, x, w_enc, b_enc):
    return pl.pallas_call(
        _tc_prep_body,
        grid=(NG,),
        in_specs=[
            pl.BlockSpec((NC, RB), lambda i: (0, i)),
            pl.BlockSpec((RB, D), lambda i: (i, 0)),
            pl.BlockSpec((D, D), lambda i: (0, 0)),
            pl.BlockSpec((1, D), lambda i: (0, 0)),
        ],
        out_specs=[
            pl.BlockSpec((RB, 1), lambda i: (i, 0)),
            pl.BlockSpec((RB, D), lambda i: (i, 0)),
        ],
        out_shape=[
            jax.ShapeDtypeStruct((NP, 1), jnp.float32),
            jax.ShapeDtypeStruct((NP, D), jnp.float32),
        ],
    )(deg_p, x, w_enc, b_enc)


def _agg(p_ref, hs_ref, dis_ref):
    agg = p_ref[0] + p_ref[1] + hs_ref[...]
    return agg * dis_ref[...]


def _tc_mid_body(p_ref, hs_ref, dis_ref, w_ref, b_ref, out_ref):
    agg = _agg(p_ref, hs_ref, dis_ref)
    h = jnp.dot(agg, w_ref[...], preferred_element_type=jnp.float32)
    out_ref[...] = jax.nn.relu(h + b_ref[...]) * dis_ref[...]


def _tc_mid(p, hs, dis, w, b):
    return pl.pallas_call(
        _tc_mid_body,
        grid=(NG,),
        in_specs=[
            pl.BlockSpec((NC, RB, D), lambda i: (0, i, 0)),
            pl.BlockSpec((RB, D), lambda i: (i, 0)),
            pl.BlockSpec((RB, 1), lambda i: (i, 0)),
            pl.BlockSpec((D, D), lambda i: (0, 0)),
            pl.BlockSpec((1, D), lambda i: (0, 0)),
        ],
        out_specs=pl.BlockSpec((RB, D), lambda i: (i, 0)),
        out_shape=jax.ShapeDtypeStruct((NP, D), jnp.float32),
    )(p, hs, dis, w, b)


def _tc_final_body(p_ref, hs_ref, dis_ref, w_ref, b_ref, wh_ref, bh_ref,
                   out_ref):
    agg = _agg(p_ref, hs_ref, dis_ref)
    h = jnp.dot(agg, w_ref[...], preferred_element_type=jnp.float32) + b_ref[...]
    out_ref[...] = (
        jnp.dot(h, wh_ref[...], preferred_element_type=jnp.float32) + bh_ref[...]
    )


def _tc_final(p, hs, dis, w, b, wh, bh):
    return pl.pallas_call(
        _tc_final_body,
        grid=(NG,),
        in_specs=[
            pl.BlockSpec((NC, RB, D), lambda i: (0, i, 0)),
            pl.BlockSpec((RB, D), lambda i: (i, 0)),
            pl.BlockSpec((RB, 1), lambda i: (i, 0)),
            pl.BlockSpec((D, D), lambda i: (0, 0)),
            pl.BlockSpec((1, D), lambda i: (0, 0)),
            pl.BlockSpec((D, D), lambda i: (0, 0)),
            pl.BlockSpec((1, D), lambda i: (0, 0)),
        ],
        out_specs=pl.BlockSpec((RB, D), lambda i: (i, 0)),
        out_shape=jax.ShapeDtypeStruct((NP, D), jnp.float32),
    )(p, hs, dis, w, b, wh, bh)


@jax.jit
def kernel(x, edge_index, W_enc, b_enc, W1, b1, W2, b2, W3, b3, W_head, b_head):
    e_pad = jnp.pad(edge_index, ((0, 0), (0, EPAD - E)), constant_values=N)
    src2d = e_pad[0].reshape(EROWS, CHUNK)
    dst2d = e_pad[1].reshape(EROWS, CHUNK)
    zeros = jnp.zeros((NP, D), jnp.float32)
    zeros1 = jnp.zeros((NP,), jnp.float32)
    x_pad = jnp.pad(x, ((0, NP - N), (0, 0)))

    deg_p = _sc_degree(dst2d, zeros1)

    dis, hs = _tc_prep(deg_p, x_pad, W_enc, b_enc.reshape(1, D))

    def scatter(hs_k):
        return _sc_scatter(hs_k, src2d, dst2d, zeros)

    p = scatter(hs)
    hs = _tc_mid(p, hs, dis, W1, b1.reshape(1, D))
    p = scatter(hs)
    hs = _tc_mid(p, hs, dis, W2, b2.reshape(1, D))
    p = scatter(hs)
    out = _tc_final(p, hs, dis, W3, b3.reshape(1, D),
                    W_head, b_head.reshape(1, D))
    return out[:N]


# speedup vs baseline: 7.1246x; 1.0681x over previous
import functools

import jax
import jax.numpy as jnp
from jax import lax
from jax.experimental import pallas as pl
from jax.experimental.pallas import tpu as pltpu
from jax.experimental.pallas import tpu_sc as plsc

N = 10000
E = 320000
D = 128
DH = D // 2

NC = 2
NS = 16
NW = NC * NS

NP = 10240
STRIPE = NP // NS

CHUNK = 128
NCHUNK = 80
BLK = 32
R0 = 128
R1 = 32
EROWS = NW * NCHUNK
EPAD = EROWS * CHUNK

_mesh = plsc.VectorSubcoreMesh(
    core_axis_name="c", subcore_axis_name="s", num_cores=NC, num_subcores=NS
)


@functools.partial(
    pl.kernel,
    out_type=jax.ShapeDtypeStruct((NC, NP), jnp.float32),
    mesh=_mesh,
    scratch_types=[
        pltpu.VMEM((NCHUNK, CHUNK), jnp.int32),
        pltpu.VMEM((CHUNK,), jnp.float32),
        pltpu.VMEM_SHARED((NP,), jnp.float32),
    ],
)
def _sc_degree(dst2d_hbm, zeros1_hbm, out_hbm, didx_v, ones_v, deg_sh):
    cid = lax.axis_index("c")
    sid = lax.axis_index("s")
    wid = cid * NS + sid
    ones = jnp.ones((16,), jnp.float32)

    @pl.loop(0, CHUNK // 16)
    def _fill(t):
        ones_v[pl.ds(t * 16, 16)] = ones

    pltpu.sync_copy(zeros1_hbm.at[pl.ds(sid * STRIPE, STRIPE)],
                    deg_sh.at[pl.ds(sid * STRIPE, STRIPE)])
    pltpu.sync_copy(dst2d_hbm.at[pl.ds(wid * NCHUNK, NCHUNK)], didx_v)

    plsc.subcore_barrier()

    @pl.loop(0, NCHUNK)
    def _accum(j):
        pltpu.sync_copy(ones_v, deg_sh.at[didx_v.at[j]], add=True)

    plsc.subcore_barrier()

    pltpu.sync_copy(deg_sh.at[pl.ds(sid * STRIPE, STRIPE)],
                    out_hbm.at[cid, pl.ds(sid * STRIPE, STRIPE)])


@functools.partial(
    pl.kernel,
    out_type=jax.ShapeDtypeStruct((NC, NP, D), jnp.float32),
    mesh=_mesh,
    scratch_types=[
        pltpu.VMEM((BLK, CHUNK), jnp.int32),
        pltpu.VMEM((BLK, CHUNK), jnp.int32),
        pltpu.VMEM((CHUNK, D), jnp.float32),
        pltpu.VMEM((CHUNK, D), jnp.float32),
        pltpu.VMEM_SHARED((NP, D), jnp.float32),
        pltpu.SemaphoreType.DMA,
        pltpu.SemaphoreType.DMA,
    ],
)
def _sc_scatter(hs_hbm, src2d_hbm, dst2d_hbm, zeros_hbm, out_hbm,
                sidx_v, didx_v, rows0_v, rows1_v, acc_sh, sem0, sem1):
    cid = lax.axis_index("c")
    sid = lax.axis_index("s")

    pltpu.sync_copy(zeros_hbm.at[pl.ds(sid * STRIPE, STRIPE)],
                    acc_sh.at[pl.ds(sid * STRIPE, STRIPE)])

    plsc.subcore_barrier()

    base_rows = jnp.where(cid == 0, sid * R0, NS * R0 + sid * R1)
    nblk = jnp.where(cid == 0, R0 // BLK, R1 // BLK)

    @pl.loop(0, nblk)
    def _block(t):
        base = base_rows + t * BLK
        pltpu.sync_copy(src2d_hbm.at[pl.ds(base, BLK)], sidx_v)
        pltpu.sync_copy(dst2d_hbm.at[pl.ds(base, BLK)], didx_v)

        pltpu.async_copy(hs_hbm.at[sidx_v.at[0]], rows0_v, sem0)

        @pl.loop(0, BLK // 2 - 1)
        def _edges(i):
            j = 2 * i
            pltpu.async_copy(hs_hbm.at[sidx_v.at[j + 1]], rows1_v, sem1)
            pltpu.make_async_copy(hs_hbm.at[sidx_v.at[j]], rows0_v,
                                  sem0).wait()
            pltpu.sync_copy(rows0_v, acc_sh.at[didx_v.at[j]], add=True)
            pltpu.async_copy(hs_hbm.at[sidx_v.at[j + 2]], rows0_v, sem0)
            pltpu.make_async_copy(hs_hbm.at[sidx_v.at[j + 1]], rows1_v,
                                  sem1).wait()
            pltpu.sync_copy(rows1_v, acc_sh.at[didx_v.at[j + 1]], add=True)

        jl = BLK - 2
        pltpu.async_copy(hs_hbm.at[sidx_v.at[jl + 1]], rows1_v, sem1)
        pltpu.make_async_copy(hs_hbm.at[sidx_v.at[jl]], rows0_v, sem0).wait()
        pltpu.sync_copy(rows0_v, acc_sh.at[didx_v.at[jl]], add=True)
        pltpu.make_async_copy(hs_hbm.at[sidx_v.at[jl + 1]], rows1_v,
                              sem1).wait()
        pltpu.sync_copy(rows1_v, acc_sh.at[didx_v.at[jl + 1]], add=True)

    plsc.subcore_barrier()

    pltpu.sync_copy(acc_sh.at[pl.ds(sid * STRIPE, STRIPE)],
                    out_hbm.at[cid, pl.ds(sid * STRIPE, STRIPE)])


RB = 512
NG = NP // RB


def _tc_prep_body(deg_ref, x_ref, w_ref, b_ref, dis_ref, hs_ref):
    cnt = jnp.sum(deg_ref[...], axis=0)
    dis = lax.rsqrt(cnt + 1.0).reshape(RB, 1)
    dis_ref[...] = dis
    h = jnp.dot(x_ref[...], w_ref[...], preferred_element_type=jnp.float32)
    hs_ref[...] = (h + b_ref[...]) * dis


def _tc_prep(deg_p, x, w_enc, b_enc):
    return pl.pallas_call(
        _tc_prep_body,
        grid=(NG,),
        in_specs=[
            pl.BlockSpec((NC, RB), lambda i: (0, i)),
            pl.BlockSpec((RB, D), lambda i: (i, 0)),
            pl.BlockSpec((D, D), lambda i: (0, 0)),
            pl.BlockSpec((1, D), lambda i: (0, 0)),
        ],
        out_specs=[
            pl.BlockSpec((RB, 1), lambda i: (i, 0)),
            pl.BlockSpec((RB, D), lambda i: (i, 0)),
        ],
        out_shape=[
            jax.ShapeDtypeStruct((NP, 1), jnp.float32),
            jax.ShapeDtypeStruct((NP, D), jnp.float32),
        ],
    )(deg_p, x, w_enc, b_enc)


def _agg(p_ref, hs_ref, dis_ref):
    agg = p_ref[0] + p_ref[1] + hs_ref[...]
    return agg * dis_ref[...]


def _tc_mid_body(p_ref, hs_ref, dis_ref, w_ref, b_ref, out_ref):
    agg = _agg(p_ref, hs_ref, dis_ref)
    h = jnp.dot(agg, w_ref[...], preferred_element_type=jnp.float32)
    out_ref[...] = jax.nn.relu(h + b_ref[...]) * dis_ref[...]


def _tc_mid(p, hs, dis, w, b):
    return pl.pallas_call(
        _tc_mid_body,
        grid=(NG,),
        in_specs=[
            pl.BlockSpec((NC, RB, D), lambda i: (0, i, 0)),
            pl.BlockSpec((RB, D), lambda i: (i, 0)),
            pl.BlockSpec((RB, 1), lambda i: (i, 0)),
            pl.BlockSpec((D, D), lambda i: (0, 0)),
            pl.BlockSpec((1, D), lambda i: (0, 0)),
        ],
        out_specs=pl.BlockSpec((RB, D), lambda i: (i, 0)),
        out_shape=jax.ShapeDtypeStruct((NP, D), jnp.float32),
    )(p, hs, dis, w, b)


def _tc_final_body(p_ref, hs_ref, dis_ref, w_ref, b_ref, wh_ref, bh_ref,
                   out_ref):
    agg = _agg(p_ref, hs_ref, dis_ref)
    h = jnp.dot(agg, w_ref[...], preferred_element_type=jnp.float32) + b_ref[...]
    out_ref[...] = (
        jnp.dot(h, wh_ref[...], preferred_element_type=jnp.float32) + bh_ref[...]
    )


def _tc_final(p, hs, dis, w, b, wh, bh):
    return pl.pallas_call(
        _tc_final_body,
        grid=(NG,),
        in_specs=[
            pl.BlockSpec((NC, RB, D), lambda i: (0, i, 0)),
            pl.BlockSpec((RB, D), lambda i: (i, 0)),
            pl.BlockSpec((RB, 1), lambda i: (i, 0)),
            pl.BlockSpec((D, D), lambda i: (0, 0)),
            pl.BlockSpec((1, D), lambda i: (0, 0)),
            pl.BlockSpec((D, D), lambda i: (0, 0)),
            pl.BlockSpec((1, D), lambda i: (0, 0)),
        ],
        out_specs=pl.BlockSpec((RB, D), lambda i: (i, 0)),
        out_shape=jax.ShapeDtypeStruct((NP, D), jnp.float32),
    )(p, hs, dis, w, b, wh, bh)


@jax.jit
def kernel(x, edge_index, W_enc, b_enc, W1, b1, W2, b2, W3, b3, W_head, b_head):
    e_pad = jnp.pad(edge_index, ((0, 0), (0, EPAD - E)), constant_values=N)
    src2d = e_pad[0].reshape(EROWS, CHUNK)
    dst2d = e_pad[1].reshape(EROWS, CHUNK)
    zeros = jnp.zeros((NP, D), jnp.float32)
    zeros1 = jnp.zeros((NP,), jnp.float32)
    x_pad = jnp.pad(x, ((0, NP - N), (0, 0)))

    deg_p = _sc_degree(dst2d, zeros1)

    dis, hs = _tc_prep(deg_p, x_pad, W_enc, b_enc.reshape(1, D))

    def scatter(hs_k):
        return _sc_scatter(hs_k, src2d, dst2d, zeros)

    p = scatter(hs)
    hs = _tc_mid(p, hs, dis, W1, b1.reshape(1, D))
    p = scatter(hs)
    hs = _tc_mid(p, hs, dis, W2, b2.reshape(1, D))
    p = scatter(hs)
    out = _tc_final(p, hs, dis, W3, b3.reshape(1, D),
                    W_head, b_head.reshape(1, D))
    return out[:N]
